# baseline (speedup 1.0000x reference)
import functools

import jax
import jax.numpy as jnp
from jax import lax
from jax.experimental import pallas as pl
from jax.experimental.pallas import tpu as pltpu
from jax.experimental.pallas import tpu_sc as plsc

NUM_BINS = 8
CMAX = 3.0
H_ = 260
W_ = 346
HW = H_ * W_
CH = 2 * NUM_BINS

NC = 2
NS = 16
L = 16

_CLIP_HI = 1.0 - 1e-06

CT = (W_ + 127) // 128
NEG_STRIDE = CT * 1024
Y_STRIDE = 2 * NEG_STRIDE
FRAME_WORDS = H_ * Y_STRIDE

SCAT_COLS = 128


def _sc_histogram(x, y, t, p, start_b, dur_b, F, N):
    C = N // NS
    FPC = F // NC
    SL = FRAME_WORDS // NS
    ZCH = 2048
    nz = -(-SL // ZCH)
    zsizes = [ZCH] * (SL // ZCH) + ([SL % ZCH] if SL % ZCH else [])
    scat_rows = C // SCAT_COLS

    mesh = plsc.VectorSubcoreMesh(core_axis_name="c", subcore_axis_name="s")

    @functools.partial(
        pl.kernel,
        out_type=jax.ShapeDtypeStruct((F * FRAME_WORDS,), jnp.float32),
        mesh=mesh,
        scratch_types=[
            pltpu.VMEM((C,), jnp.int32),
            pltpu.VMEM((C,), jnp.int32),
            pltpu.VMEM((C,), jnp.float32),
            pltpu.VMEM((C,), jnp.float32),
            pltpu.VMEM((L,), jnp.float32),
            pltpu.VMEM((L,), jnp.float32),
            pltpu.VMEM((scat_rows, SCAT_COLS), jnp.int32),
            pltpu.VMEM((scat_rows, SCAT_COLS), jnp.float32),
            pltpu.VMEM((ZCH,), jnp.float32),
            pltpu.VMEM((ZCH,), jnp.float32),
            pltpu.VMEM((ZCH,), jnp.float32),
            pltpu.VMEM_SHARED((FRAME_WORDS,), jnp.float32),
            pltpu.SemaphoreType.DMA,
            pltpu.SemaphoreType.DMA,
            pltpu.SemaphoreType.DMA,
            pltpu.SemaphoreType.DMA,
            pltpu.SemaphoreType.DMA,
        ],
    )
    def hist_kernel(x_h, y_h, t_h, p_h, sb_h, db_h, out_h,
                    x_v, y_v, t_v, p_v, s_v, d_v, idx_v, w_v, z_v,
                    o_a, o_b, hist, sem_e, sem_z, sem_s, sem_g, sem_w):
        cid = lax.axis_index("c")
        sid = lax.axis_index("s")

        def zinit(i, _):
            z_v[pl.ds(i * L, L)] = jnp.zeros((L,), jnp.float32)
            return 0
        lax.fori_loop(0, ZCH // L, zinit, 0)

        zds = []
        off = 0
        for sz in zsizes:
            zds.append(pltpu.async_copy(
                z_v.at[pl.ds(0, sz)],
                hist.at[pl.ds(sid * SL + off, sz)], sem_z))
            off += sz
        for dsc in zds:
            dsc.wait()

        def frame_body(fl, _):
            f = cid * FPC + fl
            hbase = sid * SL

            eoff = f * N + sid * C
            evs = [
                pltpu.async_copy(x_h.at[pl.ds(eoff, C)], x_v, sem_e),
                pltpu.async_copy(y_h.at[pl.ds(eoff, C)], y_v, sem_e),
                pltpu.async_copy(t_h.at[pl.ds(eoff, C)], t_v, sem_e),
                pltpu.async_copy(p_h.at[pl.ds(eoff, C)], p_v, sem_e),
                pltpu.async_copy(sb_h.at[pl.ds(f * L, L)], s_v, sem_e),
                pltpu.async_copy(db_h.at[pl.ds(f * L, L)], d_v, sem_e),
            ]
            for dsc in evs:
                dsc.wait()
            sv = s_v[...]
            dv = d_v[...]

            def chunk(j, _):
                def sub(k, _):
                    o = j * SCAT_COLS + k * L
                    xv = jnp.clip(x_v[pl.ds(o, L)], 0, W_ - 1)
                    yv = jnp.clip(y_v[pl.ds(o, L)], 0, H_ - 1)
                    tv = t_v[pl.ds(o, L)]
                    pv = p_v[pl.ds(o, L)]
                    q = jnp.clip((tv - sv) / dv, 0.0, _CLIP_HI)
                    b = jnp.minimum(
                        (q * float(NUM_BINS)).astype(jnp.int32),
                        NUM_BINS - 1)
                    neg = jnp.where(pv > 0.0, 0, 1).astype(jnp.int32)
                    idx_v[j, pl.ds(k * L, L)] = (
                        yv * Y_STRIDE
                        + neg * NEG_STRIDE
                        + lax.shift_right_logical(xv, 7) * 1024
                        + b * 128
                        + jnp.bitwise_and(xv, 127))
                    w_v[j, pl.ds(k * L, L)] = jnp.abs(pv)
                    return 0
                lax.fori_loop(0, SCAT_COLS // L, sub, 0)
                return 0
            lax.fori_loop(0, scat_rows, chunk, 0)

            plsc.subcore_barrier()

            sds = [
                pltpu.async_copy(w_v.at[j], hist.at[idx_v.at[j]], sem_s,
                                 add=True)
                for j in range(scat_rows)
            ]
            for dsc in sds:
                dsc.wait()

            plsc.subcore_barrier()

            obase = f * FRAME_WORDS + hbase
            offs = []
            off = 0
            for sz in zsizes:
                offs.append((off, sz))
                off += sz
            bufs = [o_a, o_b]
            gd = [None] * nz
            wd = [None] * nz
            rz = [None] * nz
            gd[0] = pltpu.async_copy(
                hist.at[pl.ds(hbase + offs[0][0], offs[0][1])],
                bufs[0].at[pl.ds(0, offs[0][1])], sem_g)
            for i, (o0, sz) in enumerate(offs):
                gd[i].wait()
                wd[i] = pltpu.async_copy(
                    bufs[i % 2].at[pl.ds(0, sz)],
                    out_h.at[pl.ds(obase + o0, sz)], sem_w)
                rz[i] = pltpu.async_copy(
                    z_v.at[pl.ds(0, sz)],
                    hist.at[pl.ds(hbase + o0, sz)], sem_z)
                if i + 1 < nz:
                    if i >= 1:
                        wd[i - 1].wait()
                    o1, sz1 = offs[i + 1]
                    gd[i + 1] = pltpu.async_copy(
                        hist.at[pl.ds(hbase + o1, sz1)],
                        bufs[(i + 1) % 2].at[pl.ds(0, sz1)], sem_g)
            wd[nz - 2].wait()
            wd[nz - 1].wait()
            for dsc in rz:
                dsc.wait()
            return 0

        lax.fori_loop(0, FPC, frame_body, 0)

    return hist_kernel(x, y, t, p, start_b, dur_b)


def _tc_normalize(raw_flat, B, S):
    wrem = W_ - (CT - 1) * 128

    def body(x_ref, o_ref):
        def tile(k):
            off = pl.multiple_of(k * 1024, 1024)
            v = x_ref[pl.ds(off, 1024)].reshape(8, 128)
            v = jnp.minimum(v, jnp.float32(CMAX))
            return jnp.log1p(v) / jnp.log1p(jnp.float32(CMAX))

        def row(y, _):
            k = y * (2 * CT)
            o_ref[0, 0, y, 0:8, 0:128] = tile(k)
            o_ref[0, 0, y, 0:8, 128:256] = tile(k + 1)
            o_ref[0, 0, y, 0:8, 256:W_] = tile(k + 2)[:, :wrem]
            o_ref[0, 0, y, 8:16, 0:128] = tile(k + 3)
            o_ref[0, 0, y, 8:16, 128:256] = tile(k + 4)
            o_ref[0, 0, y, 8:16, 256:W_] = tile(k + 5)[:, :wrem]
            return 0
        lax.fori_loop(0, H_, row, 0, unroll=4)

    return pl.pallas_call(
        body,
        grid=(B * S,),
        in_specs=[pl.BlockSpec((FRAME_WORDS,), lambda g: (g,))],
        out_specs=pl.BlockSpec(
            (1, 1, H_, CH, W_),
            lambda g: (g // S, g % S, 0, 0, 0)),
        out_shape=jax.ShapeDtypeStruct((B, S, H_, CH, W_), jnp.float32),
    )(raw_flat)


def kernel(event_xy, event_t, event_p, event_time_range, height, width):
    del height, width
    B, S, N = event_t.shape
    F = B * S

    x = event_xy[..., 0].reshape(F * N)
    y = event_xy[..., 1].reshape(F * N)
    t = event_t.reshape(F * N)
    p = event_p.reshape(F * N)

    start = event_time_range[..., 0].reshape(F)
    dur = jnp.maximum(event_time_range[..., 1].reshape(F) - start, 1.0)
    start_b = jnp.broadcast_to(start[:, None], (F, L)).reshape(F * L)
    dur_b = jnp.broadcast_to(dur[:, None], (F, L)).reshape(F * L)

    raw = _sc_histogram(x, y, t, p, start_b, dur_b, F, N)
    out = _tc_normalize(raw, B, S)
    return jnp.transpose(out, (0, 1, 3, 2, 4))


# speedup vs baseline: 1.1112x; 1.1112x over previous
import functools

import jax
import jax.numpy as jnp
from jax import lax
from jax.experimental import pallas as pl
from jax.experimental.pallas import tpu as pltpu
from jax.experimental.pallas import tpu_sc as plsc

NUM_BINS = 8
CMAX = 3.0
H_ = 260
W_ = 346
HW = H_ * W_
CH = 2 * NUM_BINS

NC = 2
NS = 16
L = 16

_CLIP_HI = 1.0 - 1e-06

CT = (W_ + 127) // 128
NEG_STRIDE = CT * 1024
Y_STRIDE = 2 * NEG_STRIDE
FRAME_WORDS = H_ * Y_STRIDE

SCAT_COLS = 128


def _sc_histogram(xy, t, p, start_b, dur_b, F, N):
    C = N // NS
    FPC = F // NC
    SL = FRAME_WORDS // NS
    ZCH = 2048
    nz = -(-SL // ZCH)
    zsizes = [ZCH] * (SL // ZCH) + ([SL % ZCH] if SL % ZCH else [])
    scat_rows = C // SCAT_COLS

    mesh = plsc.VectorSubcoreMesh(core_axis_name="c", subcore_axis_name="s")

    @functools.partial(
        pl.kernel,
        out_type=jax.ShapeDtypeStruct((F * FRAME_WORDS,), jnp.float32),
        mesh=mesh,
        scratch_types=[
            pltpu.VMEM((2 * C,), jnp.int32),
            pltpu.VMEM((C,), jnp.float32),
            pltpu.VMEM((C,), jnp.float32),
            pltpu.VMEM((L,), jnp.float32),
            pltpu.VMEM((L,), jnp.float32),
            pltpu.VMEM((scat_rows, SCAT_COLS), jnp.int32),
            pltpu.VMEM((scat_rows, SCAT_COLS), jnp.float32),
            pltpu.VMEM((ZCH,), jnp.float32),
            pltpu.VMEM((ZCH,), jnp.float32),
            pltpu.VMEM((ZCH,), jnp.float32),
            pltpu.VMEM_SHARED((FRAME_WORDS,), jnp.float32),
            pltpu.SemaphoreType.DMA,
            pltpu.SemaphoreType.DMA,
            pltpu.SemaphoreType.DMA,
            pltpu.SemaphoreType.DMA,
            pltpu.SemaphoreType.DMA,
        ],
    )
    def hist_kernel(xy_h, t_h, p_h, sb_h, db_h, out_h,
                    xy_v, t_v, p_v, s_v, d_v, idx_v, w_v, z_v,
                    o_a, o_b, hist, sem_e, sem_z, sem_s, sem_g, sem_w):
        cid = lax.axis_index("c")
        sid = lax.axis_index("s")

        def zinit(i, _):
            z_v[pl.ds(i * L, L)] = jnp.zeros((L,), jnp.float32)
            return 0
        lax.fori_loop(0, ZCH // L, zinit, 0)

        def frame_body(fl, _):
            f = cid * FPC + fl
            hbase = sid * SL

            eoff = f * N + sid * C
            evs = [
                pltpu.async_copy(xy_h.at[pl.ds(2 * eoff, 2 * C)], xy_v,
                                 sem_e),
                pltpu.async_copy(t_h.at[pl.ds(eoff, C)], t_v, sem_e),
                pltpu.async_copy(p_h.at[pl.ds(eoff, C)], p_v, sem_e),
                pltpu.async_copy(sb_h.at[pl.ds(f * L, L)], s_v, sem_e),
                pltpu.async_copy(db_h.at[pl.ds(f * L, L)], d_v, sem_e),
            ]
            zds = []
            off = 0
            for sz in zsizes:
                zds.append(pltpu.async_copy(
                    z_v.at[pl.ds(0, sz)],
                    hist.at[pl.ds(hbase + off, sz)], sem_z))
                off += sz
            for dsc in evs:
                dsc.wait()
            sv = s_v[...]
            dv = d_v[...]

            def chunk(j, _):
                def sub(k, _):
                    o = j * SCAT_COLS + k * L
                    xv = jnp.clip(
                        xy_v[pl.ds(j * (2 * SCAT_COLS) + k * L, L)],
                        0, W_ - 1)
                    yv = jnp.clip(
                        xy_v[pl.ds(j * (2 * SCAT_COLS) + SCAT_COLS + k * L,
                                   L)],
                        0, H_ - 1)
                    tv = t_v[pl.ds(o, L)]
                    pv = p_v[pl.ds(o, L)]
                    q = jnp.clip((tv - sv) / dv, 0.0, _CLIP_HI)
                    b = jnp.minimum(
                        (q * float(NUM_BINS)).astype(jnp.int32),
                        NUM_BINS - 1)
                    neg = jnp.where(pv > 0.0, 0, 1).astype(jnp.int32)
                    idx_v[j, pl.ds(k * L, L)] = (
                        yv * Y_STRIDE
                        + neg * NEG_STRIDE
                        + lax.shift_right_logical(xv, 7) * 1024
                        + b * 128
                        + jnp.bitwise_and(xv, 127))
                    w_v[j, pl.ds(k * L, L)] = jnp.abs(pv)
                    return 0
                lax.fori_loop(0, SCAT_COLS // L, sub, 0)
                return 0
            lax.fori_loop(0, scat_rows, chunk, 0)

            for dsc in zds:
                dsc.wait()
            plsc.subcore_barrier()

            sds = [
                pltpu.async_copy(w_v.at[j], hist.at[idx_v.at[j]], sem_s,
                                 add=True)
                for j in range(scat_rows)
            ]
            for dsc in sds:
                dsc.wait()

            plsc.subcore_barrier()

            obase = f * FRAME_WORDS + hbase
            offs = []
            off = 0
            for sz in zsizes:
                offs.append((off, sz))
                off += sz
            bufs = [o_a, o_b]
            gd = [None] * nz
            wd = [None] * nz
            gd[0] = pltpu.async_copy(
                hist.at[pl.ds(hbase + offs[0][0], offs[0][1])],
                bufs[0].at[pl.ds(0, offs[0][1])], sem_g)
            for i, (o0, sz) in enumerate(offs):
                gd[i].wait()
                wd[i] = pltpu.async_copy(
                    bufs[i % 2].at[pl.ds(0, sz)],
                    out_h.at[pl.ds(obase + o0, sz)], sem_w)
                if i + 1 < nz:
                    if i >= 1:
                        wd[i - 1].wait()
                    o1, sz1 = offs[i + 1]
                    gd[i + 1] = pltpu.async_copy(
                        hist.at[pl.ds(hbase + o1, sz1)],
                        bufs[(i + 1) % 2].at[pl.ds(0, sz1)], sem_g)
            wd[nz - 2].wait()
            wd[nz - 1].wait()
            return 0

        lax.fori_loop(0, FPC, frame_body, 0)

    return hist_kernel(xy, t, p, start_b, dur_b)


def _tc_normalize(raw_flat, B, S):
    wrem = W_ - (CT - 1) * 128

    def body(x_ref, o_ref):
        def tile(k):
            off = pl.multiple_of(k * 1024, 1024)
            v = x_ref[pl.ds(off, 1024)].reshape(8, 128)
            v = jnp.minimum(v, jnp.float32(CMAX))
            return jnp.log1p(v) / jnp.log1p(jnp.float32(CMAX))

        def row(y, _):
            k = y * (2 * CT)
            o_ref[0, 0, y, 0:8, 0:128] = tile(k)
            o_ref[0, 0, y, 0:8, 128:256] = tile(k + 1)
            o_ref[0, 0, y, 0:8, 256:W_] = tile(k + 2)[:, :wrem]
            o_ref[0, 0, y, 8:16, 0:128] = tile(k + 3)
            o_ref[0, 0, y, 8:16, 128:256] = tile(k + 4)
            o_ref[0, 0, y, 8:16, 256:W_] = tile(k + 5)[:, :wrem]
            return 0
        lax.fori_loop(0, H_, row, 0, unroll=4)

    return pl.pallas_call(
        body,
        grid=(B * S,),
        in_specs=[pl.BlockSpec((FRAME_WORDS,), lambda g: (g,))],
        out_specs=pl.BlockSpec(
            (1, 1, H_, CH, W_),
            lambda g: (g // S, g % S, 0, 0, 0)),
        out_shape=jax.ShapeDtypeStruct((B, S, H_, CH, W_), jnp.float32),
    )(raw_flat)


def kernel(event_xy, event_t, event_p, event_time_range, height, width):
    del height, width
    B, S, N = event_t.shape
    F = B * S

    xy = (event_xy.reshape(B, S, N // 128, 128, 2)
          .transpose(0, 1, 2, 4, 3)
          .reshape(F * N * 2))
    t = event_t.reshape(F * N)
    p = event_p.reshape(F * N)

    start = event_time_range[..., 0].reshape(F)
    dur = jnp.maximum(event_time_range[..., 1].reshape(F) - start, 1.0)
    start_b = jnp.broadcast_to(start[:, None], (F, L)).reshape(F * L)
    dur_b = jnp.broadcast_to(dur[:, None], (F, L)).reshape(F * L)

    raw = _sc_histogram(xy, t, p, start_b, dur_b, F, N)
    out = _tc_normalize(raw, B, S)
    return jnp.transpose(out, (0, 1, 3, 2, 4))


# speedup vs baseline: 1.1411x; 1.0269x over previous
import functools

import jax
import jax.numpy as jnp
from jax import lax
from jax.experimental import pallas as pl
from jax.experimental.pallas import tpu as pltpu
from jax.experimental.pallas import tpu_sc as plsc

NUM_BINS = 8
CMAX = 3.0
H_ = 260
W_ = 346
HW = H_ * W_
CH = 2 * NUM_BINS

NC = 2
NS = 16
L = 16

_CLIP_HI = 1.0 - 1e-06

CT = (W_ + 127) // 128
NEG_STRIDE = CT * 1024
Y_STRIDE = 2 * NEG_STRIDE
FRAME_WORDS = H_ * Y_STRIDE

SCAT_COLS = 128


def _sc_histogram(xy, t, p, start_b, dur_b, F, N, SPF):
    C = N // NS
    FPC = F // NC
    SL = FRAME_WORDS // NS
    ZCH = 2048
    nz = -(-SL // ZCH)
    zsizes = [ZCH] * (SL // ZCH) + ([SL % ZCH] if SL % ZCH else [])
    scat_rows = C // SCAT_COLS

    mesh = plsc.VectorSubcoreMesh(core_axis_name="c", subcore_axis_name="s")

    @functools.partial(
        pl.kernel,
        out_type=jax.ShapeDtypeStruct((F * FRAME_WORDS,), jnp.float32),
        mesh=mesh,
        scratch_types=[
            pltpu.VMEM((2 * C,), jnp.int32),
            pltpu.VMEM((C,), jnp.float32),
            pltpu.VMEM((C,), jnp.float32),
            pltpu.VMEM((L,), jnp.float32),
            pltpu.VMEM((L,), jnp.float32),
            pltpu.VMEM((scat_rows, SCAT_COLS), jnp.int32),
            pltpu.VMEM((scat_rows, SCAT_COLS), jnp.float32),
            pltpu.VMEM((ZCH,), jnp.float32),
            pltpu.VMEM((ZCH,), jnp.float32),
            pltpu.VMEM((ZCH,), jnp.float32),
            pltpu.VMEM_SHARED((FRAME_WORDS,), jnp.float32),
            pltpu.SemaphoreType.DMA,
            pltpu.SemaphoreType.DMA,
            pltpu.SemaphoreType.DMA,
            pltpu.SemaphoreType.DMA,
            pltpu.SemaphoreType.DMA,
        ],
    )
    def hist_kernel(xy_h, t_h, p_h, sb_h, db_h, out_h,
                    xy_v, t_v, p_v, s_v, d_v, idx_v, w_v, z_v,
                    o_a, o_b, hist, sem_e, sem_z, sem_s, sem_g, sem_w):
        cid = lax.axis_index("c")
        sid = lax.axis_index("s")

        def zinit(i, _):
            z_v[pl.ds(i * L, L)] = jnp.zeros((L,), jnp.float32)
            return 0
        lax.fori_loop(0, ZCH // L, zinit, 0)

        def frame_body(fl, _):
            f = cid * FPC + fl
            hbase = sid * SL

            eoff = f * N + sid * C
            bb = f // SPF
            ss = f - bb * SPF
            evs = [
                pltpu.async_copy(xy_h.at[pl.ds(2 * eoff, 2 * C)], xy_v,
                                 sem_e),
                pltpu.async_copy(sb_h.at[pl.ds(f * L, L)], s_v, sem_e),
                pltpu.async_copy(db_h.at[pl.ds(f * L, L)], d_v, sem_e),
            ]
            for j in range(scat_rows):
                soff = bb * (SPF * N) + (sid * scat_rows + j) * 1024 + ss * 128
                evs.append(pltpu.async_copy(
                    t_h.at[pl.ds(soff, SCAT_COLS)],
                    t_v.at[pl.ds(j * SCAT_COLS, SCAT_COLS)], sem_e))
                evs.append(pltpu.async_copy(
                    p_h.at[pl.ds(soff, SCAT_COLS)],
                    p_v.at[pl.ds(j * SCAT_COLS, SCAT_COLS)], sem_e))
            zds = []
            off = 0
            for sz in zsizes:
                zds.append(pltpu.async_copy(
                    z_v.at[pl.ds(0, sz)],
                    hist.at[pl.ds(hbase + off, sz)], sem_z))
                off += sz
            for dsc in evs:
                dsc.wait()
            sv = s_v[...]
            dv = d_v[...]

            def chunk(j, _):
                def sub(k, _):
                    o = j * SCAT_COLS + k * L
                    xv = jnp.clip(
                        xy_v[pl.ds(j * (2 * SCAT_COLS) + k * L, L)],
                        0, W_ - 1)
                    yv = jnp.clip(
                        xy_v[pl.ds(j * (2 * SCAT_COLS) + SCAT_COLS + k * L,
                                   L)],
                        0, H_ - 1)
                    tv = t_v[pl.ds(o, L)]
                    pv = p_v[pl.ds(o, L)]
                    q = jnp.clip((tv - sv) / dv, 0.0, _CLIP_HI)
                    b = jnp.minimum(
                        (q * float(NUM_BINS)).astype(jnp.int32),
                        NUM_BINS - 1)
                    neg = jnp.where(pv > 0.0, 0, 1).astype(jnp.int32)
                    idx_v[j, pl.ds(k * L, L)] = (
                        yv * Y_STRIDE
                        + neg * NEG_STRIDE
                        + lax.shift_right_logical(xv, 7) * 1024
                        + b * 128
                        + jnp.bitwise_and(xv, 127))
                    w_v[j, pl.ds(k * L, L)] = jnp.abs(pv)
                    return 0
                lax.fori_loop(0, SCAT_COLS // L, sub, 0)
                return 0
            lax.fori_loop(0, scat_rows, chunk, 0)

            for dsc in zds:
                dsc.wait()
            plsc.subcore_barrier()

            sds = [
                pltpu.async_copy(w_v.at[j], hist.at[idx_v.at[j]], sem_s,
                                 add=True)
                for j in range(scat_rows)
            ]
            for dsc in sds:
                dsc.wait()

            plsc.subcore_barrier()

            obase = f * FRAME_WORDS + hbase
            offs = []
            off = 0
            for sz in zsizes:
                offs.append((off, sz))
                off += sz
            bufs = [o_a, o_b]
            gd = [None] * nz
            wd = [None] * nz
            gd[0] = pltpu.async_copy(
                hist.at[pl.ds(hbase + offs[0][0], offs[0][1])],
                bufs[0].at[pl.ds(0, offs[0][1])], sem_g)
            for i, (o0, sz) in enumerate(offs):
                gd[i].wait()
                wd[i] = pltpu.async_copy(
                    bufs[i % 2].at[pl.ds(0, sz)],
                    out_h.at[pl.ds(obase + o0, sz)], sem_w)
                if i + 1 < nz:
                    if i >= 1:
                        wd[i - 1].wait()
                    o1, sz1 = offs[i + 1]
                    gd[i + 1] = pltpu.async_copy(
                        hist.at[pl.ds(hbase + o1, sz1)],
                        bufs[(i + 1) % 2].at[pl.ds(0, sz1)], sem_g)
            wd[nz - 2].wait()
            wd[nz - 1].wait()
            return 0

        lax.fori_loop(0, FPC, frame_body, 0)

    return hist_kernel(xy, t, p, start_b, dur_b)


def _tc_normalize(raw_flat, B, S):
    wrem = W_ - (CT - 1) * 128

    def body(x_ref, o_ref):
        def tile(k):
            off = pl.multiple_of(k * 1024, 1024)
            v = x_ref[pl.ds(off, 1024)].reshape(8, 128)
            v = jnp.minimum(v, jnp.float32(CMAX))
            return jnp.log1p(v) / jnp.log1p(jnp.float32(CMAX))

        def row(y, _):
            k = y * (2 * CT)
            o_ref[0, 0, y, 0:8, 0:128] = tile(k)
            o_ref[0, 0, y, 0:8, 128:256] = tile(k + 1)
            o_ref[0, 0, y, 0:8, 256:W_] = tile(k + 2)[:, :wrem]
            o_ref[0, 0, y, 8:16, 0:128] = tile(k + 3)
            o_ref[0, 0, y, 8:16, 128:256] = tile(k + 4)
            o_ref[0, 0, y, 8:16, 256:W_] = tile(k + 5)[:, :wrem]
            return 0
        lax.fori_loop(0, H_, row, 0, unroll=4)

    return pl.pallas_call(
        body,
        grid=(B * S,),
        in_specs=[pl.BlockSpec((FRAME_WORDS,), lambda g: (g,))],
        out_specs=pl.BlockSpec(
            (1, 1, H_, CH, W_),
            lambda g: (g // S, g % S, 0, 0, 0)),
        out_shape=jax.ShapeDtypeStruct((B, S, H_, CH, W_), jnp.float32),
    )(raw_flat)


def kernel(event_xy, event_t, event_p, event_time_range, height, width):
    del height, width
    B, S, N = event_t.shape
    F = B * S

    xy = (event_xy.reshape(B, S, N // 128, 128, 2)
          .transpose(0, 1, 2, 4, 3)
          .reshape(F * N * 2))
    t = (event_t.reshape(B, S, N // 128, 128)
         .transpose(0, 2, 1, 3).reshape(F * N))
    p = (event_p.reshape(B, S, N // 128, 128)
         .transpose(0, 2, 1, 3).reshape(F * N))

    start = event_time_range[..., 0].reshape(F)
    dur = jnp.maximum(event_time_range[..., 1].reshape(F) - start, 1.0)
    start_b = jnp.broadcast_to(start[:, None], (F, L)).reshape(F * L)
    dur_b = jnp.broadcast_to(dur[:, None], (F, L)).reshape(F * L)

    raw = _sc_histogram(xy, t, p, start_b, dur_b, F, N, S)
    out = _tc_normalize(raw, B, S)
    return jnp.transpose(out, (0, 1, 3, 2, 4))


# speedup vs baseline: 1.3166x; 1.1538x over previous
import functools

import jax
import jax.numpy as jnp
from jax import lax
from jax.experimental import pallas as pl
from jax.experimental.pallas import tpu as pltpu
from jax.experimental.pallas import tpu_sc as plsc

NUM_BINS = 8
CMAX = 3.0
H_ = 260
W_ = 346
HW = H_ * W_
CH = 2 * NUM_BINS

NC = 2
NS = 16
L = 16

_CLIP_HI = 1.0 - 1e-06

CT = (W_ + 127) // 128
NEG_STRIDE = CT * 1024
Y_STRIDE = 2 * NEG_STRIDE
FRAME_WORDS = H_ * Y_STRIDE

SCAT_COLS = 128


def _sc_histogram(xy, t, p, start_b, dur_b, F, N, SPF):
    C = N // NS
    FPC = F // NC
    SL = FRAME_WORDS // NS
    ZCH = 2048
    nz = -(-SL // ZCH)
    zsizes = [ZCH] * (SL // ZCH) + ([SL % ZCH] if SL % ZCH else [])
    scat_rows = C // SCAT_COLS

    mesh = plsc.VectorSubcoreMesh(core_axis_name="c", subcore_axis_name="s")

    @functools.partial(
        pl.kernel,
        out_type=jax.ShapeDtypeStruct((F * FRAME_WORDS,), jnp.float32),
        mesh=mesh,
        scratch_types=[
            pltpu.VMEM((2 * C,), jnp.int32),
            pltpu.VMEM((C,), jnp.float32),
            pltpu.VMEM((C,), jnp.float32),
            pltpu.VMEM((L,), jnp.float32),
            pltpu.VMEM((L,), jnp.float32),
            pltpu.VMEM((scat_rows, SCAT_COLS), jnp.int32),
            pltpu.VMEM((scat_rows, SCAT_COLS), jnp.float32),
            pltpu.VMEM((ZCH,), jnp.float32),
            pltpu.VMEM((ZCH,), jnp.float32),
            pltpu.VMEM((ZCH,), jnp.float32),
            pltpu.VMEM_SHARED((FRAME_WORDS,), jnp.float32),
            pltpu.SemaphoreType.DMA,
            pltpu.SemaphoreType.DMA,
            pltpu.SemaphoreType.DMA,
            pltpu.SemaphoreType.DMA,
            pltpu.SemaphoreType.DMA,
        ],
    )
    def hist_kernel(xy_h, t_h, p_h, sb_h, db_h, out_h,
                    xy_v, t_v, p_v, s_v, d_v, idx_v, w_v, z_v,
                    o_a, o_b, hist, sem_e, sem_z, sem_s, sem_g, sem_w):
        cid = lax.axis_index("c")
        sid = lax.axis_index("s")

        def zinit(i, _):
            z_v[pl.ds(i * L, L)] = jnp.zeros((L,), jnp.float32)
            return 0
        lax.fori_loop(0, ZCH // L, zinit, 0)

        def frame_body(fl, _):
            f = cid * FPC + fl
            hbase = sid * SL

            eoff = f * N + sid * C
            bb = f // SPF
            ss = f - bb * SPF
            evs = [
                pltpu.async_copy(xy_h.at[pl.ds(2 * eoff, 2 * C)], xy_v,
                                 sem_e),
                pltpu.async_copy(sb_h.at[pl.ds(f * L, L)], s_v, sem_e),
                pltpu.async_copy(db_h.at[pl.ds(f * L, L)], d_v, sem_e),
            ]
            for j in range(scat_rows):
                soff = bb * (SPF * N) + (sid * scat_rows + j) * 1024 + ss * 128
                evs.append(pltpu.async_copy(
                    t_h.at[pl.ds(soff, SCAT_COLS)],
                    t_v.at[pl.ds(j * SCAT_COLS, SCAT_COLS)], sem_e))
                evs.append(pltpu.async_copy(
                    p_h.at[pl.ds(soff, SCAT_COLS)],
                    p_v.at[pl.ds(j * SCAT_COLS, SCAT_COLS)], sem_e))
            zds = []
            off = 0
            for sz in zsizes:
                zds.append(pltpu.async_copy(
                    z_v.at[pl.ds(0, sz)],
                    hist.at[pl.ds(hbase + off, sz)], sem_z))
                off += sz
            for dsc in evs:
                dsc.wait()
            sv = s_v[...]
            dv = d_v[...]

            def chunk(j, _):
                def sub(k, _):
                    o = j * SCAT_COLS + k * L
                    xv = jnp.clip(
                        xy_v[pl.ds(j * (2 * SCAT_COLS) + k * L, L)],
                        0, W_ - 1)
                    yv = jnp.clip(
                        xy_v[pl.ds(j * (2 * SCAT_COLS) + SCAT_COLS + k * L,
                                   L)],
                        0, H_ - 1)
                    tv = t_v[pl.ds(o, L)]
                    pv = p_v[pl.ds(o, L)]
                    q = jnp.clip((tv - sv) / dv, 0.0, _CLIP_HI)
                    b = jnp.minimum(
                        (q * float(NUM_BINS)).astype(jnp.int32),
                        NUM_BINS - 1)
                    neg = jnp.where(pv > 0.0, 0, 1).astype(jnp.int32)
                    idx_v[j, pl.ds(k * L, L)] = (
                        yv * Y_STRIDE
                        + neg * NEG_STRIDE
                        + lax.shift_right_logical(xv, 7) * 1024
                        + b * 128
                        + jnp.bitwise_and(xv, 127))
                    w_v[j, pl.ds(k * L, L)] = jnp.abs(pv)
                    return 0
                lax.fori_loop(0, SCAT_COLS // L, sub, 0)
                return 0
            lax.fori_loop(0, scat_rows, chunk, 0)

            for dsc in zds:
                dsc.wait()
            plsc.subcore_barrier()

            sds = [
                pltpu.async_copy(w_v.at[j], hist.at[idx_v.at[j]], sem_s,
                                 add=True)
                for j in range(scat_rows)
            ]
            for dsc in sds:
                dsc.wait()

            plsc.subcore_barrier()

            obase = f * FRAME_WORDS + hbase
            offs = []
            off = 0
            for sz in zsizes:
                offs.append((off, sz))
                off += sz
            KB = 4
            bufs = [o_a, o_b, t_v.at[pl.ds(0, ZCH)],
                    p_v.at[pl.ds(0, ZCH)]]
            gd = [None] * nz
            wd = [None] * nz
            for i in range(min(2, nz)):
                o0, sz = offs[i]
                gd[i] = pltpu.async_copy(
                    hist.at[pl.ds(hbase + o0, sz)],
                    bufs[i % KB].at[pl.ds(0, sz)], sem_g)
            for i, (o0, sz) in enumerate(offs):
                gd[i].wait()
                wd[i] = pltpu.async_copy(
                    bufs[i % KB].at[pl.ds(0, sz)],
                    out_h.at[pl.ds(obase + o0, sz)], sem_w)
                if i + 2 < nz:
                    if i >= 2:
                        wd[i - 2].wait()
                    o1, sz1 = offs[i + 2]
                    gd[i + 2] = pltpu.async_copy(
                        hist.at[pl.ds(hbase + o1, sz1)],
                        bufs[(i + 2) % KB].at[pl.ds(0, sz1)], sem_g)
            for i in range(max(0, nz - KB), nz):
                wd[i].wait()
            return 0

        lax.fori_loop(0, FPC, frame_body, 0)

    return hist_kernel(xy, t, p, start_b, dur_b)


def _tc_normalize(raw_flat, B, S):
    wrem = W_ - (CT - 1) * 128

    def body(x_ref, o_ref):
        def tile(k):
            off = pl.multiple_of(k * 1024, 1024)
            v = x_ref[pl.ds(off, 1024)].reshape(8, 128)
            v = jnp.minimum(v, jnp.float32(CMAX))
            return jnp.log1p(v) / jnp.log1p(jnp.float32(CMAX))

        def row(y, _):
            k = y * (2 * CT)
            o_ref[0, 0, y, 0:8, 0:128] = tile(k)
            o_ref[0, 0, y, 0:8, 128:256] = tile(k + 1)
            o_ref[0, 0, y, 0:8, 256:W_] = tile(k + 2)[:, :wrem]
            o_ref[0, 0, y, 8:16, 0:128] = tile(k + 3)
            o_ref[0, 0, y, 8:16, 128:256] = tile(k + 4)
            o_ref[0, 0, y, 8:16, 256:W_] = tile(k + 5)[:, :wrem]
            return 0
        lax.fori_loop(0, H_, row, 0, unroll=4)

    return pl.pallas_call(
        body,
        grid=(B * S,),
        in_specs=[pl.BlockSpec((FRAME_WORDS,), lambda g: (g,))],
        out_specs=pl.BlockSpec(
            (1, 1, H_, CH, W_),
            lambda g: (g // S, g % S, 0, 0, 0)),
        out_shape=jax.ShapeDtypeStruct((B, S, H_, CH, W_), jnp.float32),
    )(raw_flat)


def kernel(event_xy, event_t, event_p, event_time_range, height, width):
    del height, width
    B, S, N = event_t.shape
    F = B * S

    xy = (event_xy.reshape(B, S, N // 128, 128, 2)
          .transpose(0, 1, 2, 4, 3)
          .reshape(F * N * 2))
    t = (event_t.reshape(B, S, N // 128, 128)
         .transpose(0, 2, 1, 3).reshape(F * N))
    p = (event_p.reshape(B, S, N // 128, 128)
         .transpose(0, 2, 1, 3).reshape(F * N))

    start = event_time_range[..., 0].reshape(F)
    dur = jnp.maximum(event_time_range[..., 1].reshape(F) - start, 1.0)
    start_b = jnp.broadcast_to(start[:, None], (F, L)).reshape(F * L)
    dur_b = jnp.broadcast_to(dur[:, None], (F, L)).reshape(F * L)

    raw = _sc_histogram(xy, t, p, start_b, dur_b, F, N, S)
    out = _tc_normalize(raw, B, S)
    return jnp.transpose(out, (0, 1, 3, 2, 4))


# speedup vs baseline: 1.3304x; 1.0104x over previous
import functools

import jax
import jax.numpy as jnp
from jax import lax
from jax.experimental import pallas as pl
from jax.experimental.pallas import tpu as pltpu
from jax.experimental.pallas import tpu_sc as plsc

NUM_BINS = 8
CMAX = 3.0
H_ = 260
W_ = 346
HW = H_ * W_
CH = 2 * NUM_BINS

NC = 2
NS = 16
L = 16

_CLIP_HI = 1.0 - 1e-06

CT = (W_ + 127) // 128
NEG_STRIDE = CT * 1024
Y_STRIDE = 2 * NEG_STRIDE
FRAME_WORDS = H_ * Y_STRIDE

SCAT_COLS = 128


def _sc_histogram(xy, t, p, start_b, dur_b, F, N, SPF):
    C = N // NS
    FPC = F // NC
    SL = FRAME_WORDS // NS
    ZCH = 2048
    nz = -(-SL // ZCH)
    zsizes = [ZCH] * (SL // ZCH) + ([SL % ZCH] if SL % ZCH else [])
    scat_rows = C // SCAT_COLS

    mesh = plsc.VectorSubcoreMesh(core_axis_name="c", subcore_axis_name="s")

    @functools.partial(
        pl.kernel,
        out_type=jax.ShapeDtypeStruct((F * FRAME_WORDS,), jnp.float32),
        mesh=mesh,
        scratch_types=[
            pltpu.VMEM((2 * C,), jnp.int32),
            pltpu.VMEM((C,), jnp.float32),
            pltpu.VMEM((C,), jnp.float32),
            pltpu.VMEM((L,), jnp.float32),
            pltpu.VMEM((L,), jnp.float32),
            pltpu.VMEM((scat_rows, SCAT_COLS), jnp.int32),
            pltpu.VMEM((scat_rows, SCAT_COLS), jnp.float32),
            pltpu.VMEM((ZCH,), jnp.float32),
            pltpu.VMEM((ZCH,), jnp.float32),
            pltpu.VMEM((ZCH,), jnp.float32),
            pltpu.VMEM_SHARED((FRAME_WORDS,), jnp.float32),
            pltpu.SemaphoreType.DMA,
            pltpu.SemaphoreType.DMA,
            pltpu.SemaphoreType.DMA,
            pltpu.SemaphoreType.DMA,
            pltpu.SemaphoreType.DMA,
        ],
    )
    def hist_kernel(xy_h, t_h, p_h, sb_h, db_h, out_h,
                    xy_v, t_v, p_v, s_v, d_v, idx_v, w_v, z_v,
                    o_a, o_b, hist, sem_e, sem_z, sem_s, sem_g, sem_w):
        cid = lax.axis_index("c")
        sid = lax.axis_index("s")

        def zinit(i, _):
            z_v[pl.ds(i * L, L)] = jnp.zeros((L,), jnp.float32)
            return 0
        lax.fori_loop(0, ZCH // L, zinit, 0)

        def frame_body(fl, _):
            f = cid * FPC + fl
            hbase = sid * SL

            eoff = f * N + sid * C
            bb = f // SPF
            ss = f - bb * SPF
            evs = [
                pltpu.async_copy(xy_h.at[pl.ds(2 * eoff, 2 * C)], xy_v,
                                 sem_e),
                pltpu.async_copy(sb_h.at[pl.ds(f * L, L)], s_v, sem_e),
                pltpu.async_copy(db_h.at[pl.ds(f * L, L)], d_v, sem_e),
            ]
            for j in range(scat_rows):
                soff = bb * (SPF * N) + (sid * scat_rows + j) * 1024 + ss * 128
                evs.append(pltpu.async_copy(
                    t_h.at[pl.ds(soff, SCAT_COLS)],
                    t_v.at[pl.ds(j * SCAT_COLS, SCAT_COLS)], sem_e))
                evs.append(pltpu.async_copy(
                    p_h.at[pl.ds(soff, SCAT_COLS)],
                    p_v.at[pl.ds(j * SCAT_COLS, SCAT_COLS)], sem_e))
            zds = []
            off = 0
            for sz in zsizes:
                zds.append(pltpu.async_copy(
                    z_v.at[pl.ds(0, sz)],
                    hist.at[pl.ds(hbase + off, sz)], sem_z))
                off += sz
            for dsc in evs:
                dsc.wait()
            sv = s_v[...]
            dv = d_v[...]

            def chunk(j, _):
                def sub(k, _):
                    o = j * SCAT_COLS + k * L
                    xv = jnp.clip(
                        xy_v[pl.ds(j * (2 * SCAT_COLS) + k * L, L)],
                        0, W_ - 1)
                    yv = jnp.clip(
                        xy_v[pl.ds(j * (2 * SCAT_COLS) + SCAT_COLS + k * L,
                                   L)],
                        0, H_ - 1)
                    tv = t_v[pl.ds(o, L)]
                    pv = p_v[pl.ds(o, L)]
                    q = jnp.clip((tv - sv) / dv, 0.0, _CLIP_HI)
                    b = jnp.minimum(
                        (q * float(NUM_BINS)).astype(jnp.int32),
                        NUM_BINS - 1)
                    neg = jnp.where(pv > 0.0, 0, 1).astype(jnp.int32)
                    idx_v[j, pl.ds(k * L, L)] = (
                        yv * Y_STRIDE
                        + neg * NEG_STRIDE
                        + lax.shift_right_logical(xv, 7) * 1024
                        + b * 128
                        + jnp.bitwise_and(xv, 127))
                    w_v[j, pl.ds(k * L, L)] = jnp.abs(pv)
                    return 0
                lax.fori_loop(0, SCAT_COLS // L, sub, 0)
                return 0
            lax.fori_loop(0, scat_rows, chunk, 0)

            for dsc in zds:
                dsc.wait()
            plsc.subcore_barrier()

            sds = [
                pltpu.async_copy(w_v.at[j], hist.at[idx_v.at[j]], sem_s,
                                 add=True)
                for j in range(scat_rows)
            ]
            for dsc in sds:
                dsc.wait()

            plsc.subcore_barrier()

            obase = f * FRAME_WORDS + hbase
            offs = []
            off = 0
            for sz in zsizes:
                offs.append((off, sz))
                off += sz
            KB = 4
            bufs = [o_a, o_b, t_v.at[pl.ds(0, ZCH)],
                    p_v.at[pl.ds(0, ZCH)]]
            gd = [None] * nz
            wd = [None] * nz
            for i in range(min(3, nz)):
                o0, sz = offs[i]
                gd[i] = pltpu.async_copy(
                    hist.at[pl.ds(hbase + o0, sz)],
                    bufs[i % KB].at[pl.ds(0, sz)], sem_g)
            for i, (o0, sz) in enumerate(offs):
                gd[i].wait()
                wd[i] = pltpu.async_copy(
                    bufs[i % KB].at[pl.ds(0, sz)],
                    out_h.at[pl.ds(obase + o0, sz)], sem_w)
                if i + 3 < nz:
                    if i >= 1:
                        wd[i - 1].wait()
                    o1, sz1 = offs[i + 3]
                    gd[i + 3] = pltpu.async_copy(
                        hist.at[pl.ds(hbase + o1, sz1)],
                        bufs[(i + 3) % KB].at[pl.ds(0, sz1)], sem_g)
            for i in range(max(0, nz - KB), nz):
                wd[i].wait()
            return 0

        lax.fori_loop(0, FPC, frame_body, 0)

    return hist_kernel(xy, t, p, start_b, dur_b)


def _tc_normalize(raw_flat, B, S):
    wrem = W_ - (CT - 1) * 128

    def body(x_ref, o_ref):
        def tile(k):
            off = pl.multiple_of(k * 1024, 1024)
            v = x_ref[pl.ds(off, 1024)].reshape(8, 128)
            v = jnp.minimum(v, jnp.float32(CMAX))
            return jnp.log1p(v) / jnp.log1p(jnp.float32(CMAX))

        def row(y, _):
            k = y * (2 * CT)
            o_ref[0, 0, y, 0:8, 0:128] = tile(k)
            o_ref[0, 0, y, 0:8, 128:256] = tile(k + 1)
            o_ref[0, 0, y, 0:8, 256:W_] = tile(k + 2)[:, :wrem]
            o_ref[0, 0, y, 8:16, 0:128] = tile(k + 3)
            o_ref[0, 0, y, 8:16, 128:256] = tile(k + 4)
            o_ref[0, 0, y, 8:16, 256:W_] = tile(k + 5)[:, :wrem]
            return 0
        lax.fori_loop(0, H_, row, 0, unroll=10)

    return pl.pallas_call(
        body,
        grid=(B * S,),
        in_specs=[pl.BlockSpec((FRAME_WORDS,), lambda g: (g,))],
        out_specs=pl.BlockSpec(
            (1, 1, H_, CH, W_),
            lambda g: (g // S, g % S, 0, 0, 0)),
        out_shape=jax.ShapeDtypeStruct((B, S, H_, CH, W_), jnp.float32),
    )(raw_flat)


def kernel(event_xy, event_t, event_p, event_time_range, height, width):
    del height, width
    B, S, N = event_t.shape
    F = B * S

    xy = (event_xy.reshape(B, S, N // 128, 128, 2)
          .transpose(0, 1, 2, 4, 3)
          .reshape(F * N * 2))
    t = (event_t.reshape(B, S, N // 128, 128)
         .transpose(0, 2, 1, 3).reshape(F * N))
    p = (event_p.reshape(B, S, N // 128, 128)
         .transpose(0, 2, 1, 3).reshape(F * N))

    start = event_time_range[..., 0].reshape(F)
    dur = jnp.maximum(event_time_range[..., 1].reshape(F) - start, 1.0)
    start_b = jnp.broadcast_to(start[:, None], (F, L)).reshape(F * L)
    dur_b = jnp.broadcast_to(dur[:, None], (F, L)).reshape(F * L)

    raw = _sc_histogram(xy, t, p, start_b, dur_b, F, N, S)
    out = _tc_normalize(raw, B, S)
    return jnp.transpose(out, (0, 1, 3, 2, 4))


# speedup vs baseline: 1.3378x; 1.0056x over previous
import functools

import jax
import jax.numpy as jnp
from jax import lax
from jax.experimental import pallas as pl
from jax.experimental.pallas import tpu as pltpu
from jax.experimental.pallas import tpu_sc as plsc

NUM_BINS = 8
CMAX = 3.0
H_ = 260
W_ = 346
HW = H_ * W_
CH = 2 * NUM_BINS

NC = 2
NS = 16
L = 16

_CLIP_HI = 1.0 - 1e-06

CT = (W_ + 127) // 128
NEG_STRIDE = CT * 1024
Y_STRIDE = 2 * NEG_STRIDE
FRAME_WORDS = H_ * Y_STRIDE

SCAT_COLS = 128


def _sc_histogram(xy, t, p, start_b, dur_b, F, N, SPF):
    C = N // NS
    FPC = F // NC
    SL = FRAME_WORDS // NS
    ZCH = 2048
    nz = -(-SL // ZCH)
    zsizes = [ZCH] * (SL // ZCH) + ([SL % ZCH] if SL % ZCH else [])
    scat_rows = C // SCAT_COLS

    mesh = plsc.VectorSubcoreMesh(core_axis_name="c", subcore_axis_name="s")

    @functools.partial(
        pl.kernel,
        out_type=jax.ShapeDtypeStruct((F * FRAME_WORDS,), jnp.float32),
        mesh=mesh,
        scratch_types=[
            pltpu.VMEM((2 * C,), jnp.int32),
            pltpu.VMEM((C,), jnp.float32),
            pltpu.VMEM((C,), jnp.float32),
            pltpu.VMEM((L,), jnp.float32),
            pltpu.VMEM((L,), jnp.float32),
            pltpu.VMEM((scat_rows, SCAT_COLS), jnp.int32),
            pltpu.VMEM((scat_rows, SCAT_COLS), jnp.float32),
            pltpu.VMEM((ZCH,), jnp.float32),
            pltpu.VMEM((ZCH,), jnp.float32),
            pltpu.VMEM((ZCH,), jnp.float32),
            pltpu.VMEM_SHARED((FRAME_WORDS,), jnp.float32),
            pltpu.SemaphoreType.DMA,
            pltpu.SemaphoreType.DMA,
            pltpu.SemaphoreType.DMA,
            pltpu.SemaphoreType.DMA,
            pltpu.SemaphoreType.DMA,
        ],
    )
    def hist_kernel(xy_h, t_h, p_h, sb_h, db_h, out_h,
                    xy_v, t_v, p_v, s_v, d_v, idx_v, w_v, z_v,
                    o_a, o_b, hist, sem_e, sem_z, sem_s, sem_g, sem_w):
        cid = lax.axis_index("c")
        sid = lax.axis_index("s")

        def zinit(i, _):
            z_v[pl.ds(i * L, L)] = jnp.zeros((L,), jnp.float32)
            return 0
        lax.fori_loop(0, ZCH // L, zinit, 0)

        def frame_body(fl, _):
            f = cid * FPC + fl
            hbase = sid * SL

            eoff = f * N + sid * C
            bb = f // SPF
            ss = f - bb * SPF
            evs = [
                pltpu.async_copy(xy_h.at[pl.ds(2 * eoff, 2 * C)], xy_v,
                                 sem_e),
                pltpu.async_copy(sb_h.at[pl.ds(f * L, L)], s_v, sem_e),
                pltpu.async_copy(db_h.at[pl.ds(f * L, L)], d_v, sem_e),
            ]
            for j in range(scat_rows):
                soff = bb * (SPF * N) + (sid * scat_rows + j) * 1024 + ss * 128
                evs.append(pltpu.async_copy(
                    t_h.at[pl.ds(soff, SCAT_COLS)],
                    t_v.at[pl.ds(j * SCAT_COLS, SCAT_COLS)], sem_e))
                evs.append(pltpu.async_copy(
                    p_h.at[pl.ds(soff, SCAT_COLS)],
                    p_v.at[pl.ds(j * SCAT_COLS, SCAT_COLS)], sem_e))
            zds = []
            off = 0
            for sz in zsizes:
                zds.append(pltpu.async_copy(
                    z_v.at[pl.ds(0, sz)],
                    hist.at[pl.ds(hbase + off, sz)], sem_z))
                off += sz
            for dsc in evs:
                dsc.wait()
            sv = s_v[...]
            dv = d_v[...]

            def chunk(j, _):
                def sub(k, _):
                    o = j * SCAT_COLS + k * L
                    xv = jnp.clip(
                        xy_v[pl.ds(j * (2 * SCAT_COLS) + k * L, L)],
                        0, W_ - 1)
                    yv = jnp.clip(
                        xy_v[pl.ds(j * (2 * SCAT_COLS) + SCAT_COLS + k * L,
                                   L)],
                        0, H_ - 1)
                    tv = t_v[pl.ds(o, L)]
                    pv = p_v[pl.ds(o, L)]
                    q = jnp.clip((tv - sv) / dv, 0.0, _CLIP_HI)
                    b = jnp.minimum(
                        (q * float(NUM_BINS)).astype(jnp.int32),
                        NUM_BINS - 1)
                    neg = jnp.where(pv > 0.0, 0, 1).astype(jnp.int32)
                    idx_v[j, pl.ds(k * L, L)] = (
                        yv * Y_STRIDE
                        + neg * NEG_STRIDE
                        + lax.shift_right_logical(xv, 7) * 1024
                        + b * 128
                        + jnp.bitwise_and(xv, 127))
                    w_v[j, pl.ds(k * L, L)] = jnp.abs(pv)
                    return 0
                lax.fori_loop(0, SCAT_COLS // L, sub, 0)
                return 0
            lax.fori_loop(0, scat_rows, chunk, 0)

            for dsc in zds:
                dsc.wait()
            plsc.subcore_barrier()

            sds = [
                pltpu.async_copy(w_v.at[j], hist.at[idx_v.at[j]], sem_s,
                                 add=True)
                for j in range(scat_rows)
            ]
            for dsc in sds:
                dsc.wait()

            plsc.subcore_barrier()

            obase = f * FRAME_WORDS + hbase
            offs = []
            off = 0
            for sz in zsizes:
                offs.append((off, sz))
                off += sz
            KB = 6
            bufs = [o_a, o_b,
                    t_v.at[pl.ds(0, ZCH)], t_v.at[pl.ds(ZCH, ZCH)],
                    p_v.at[pl.ds(0, ZCH)], p_v.at[pl.ds(ZCH, ZCH)]]
            gd = [None] * nz
            wd = [None] * nz
            for i in range(min(4, nz)):
                o0, sz = offs[i]
                gd[i] = pltpu.async_copy(
                    hist.at[pl.ds(hbase + o0, sz)],
                    bufs[i % KB].at[pl.ds(0, sz)], sem_g)
            for i, (o0, sz) in enumerate(offs):
                gd[i].wait()
                wd[i] = pltpu.async_copy(
                    bufs[i % KB].at[pl.ds(0, sz)],
                    out_h.at[pl.ds(obase + o0, sz)], sem_w)
                if i + 4 < nz:
                    if i >= 2:
                        wd[i - 2].wait()
                    o1, sz1 = offs[i + 4]
                    gd[i + 4] = pltpu.async_copy(
                        hist.at[pl.ds(hbase + o1, sz1)],
                        bufs[(i + 4) % KB].at[pl.ds(0, sz1)], sem_g)
            for i in range(max(0, nz - KB), nz):
                wd[i].wait()
            return 0

        lax.fori_loop(0, FPC, frame_body, 0)

    return hist_kernel(xy, t, p, start_b, dur_b)


def _tc_normalize(raw_flat, B, S):
    wrem = W_ - (CT - 1) * 128

    def body(x_ref, o_ref):
        def tile(k):
            off = pl.multiple_of(k * 1024, 1024)
            v = x_ref[pl.ds(off, 1024)].reshape(8, 128)
            v = jnp.minimum(v, jnp.float32(CMAX))
            return jnp.log1p(v) / jnp.log1p(jnp.float32(CMAX))

        def row(y, _):
            k = y * (2 * CT)
            o_ref[0, 0, y, 0:8, 0:128] = tile(k)
            o_ref[0, 0, y, 0:8, 128:256] = tile(k + 1)
            o_ref[0, 0, y, 0:8, 256:W_] = tile(k + 2)[:, :wrem]
            o_ref[0, 0, y, 8:16, 0:128] = tile(k + 3)
            o_ref[0, 0, y, 8:16, 128:256] = tile(k + 4)
            o_ref[0, 0, y, 8:16, 256:W_] = tile(k + 5)[:, :wrem]
            return 0
        lax.fori_loop(0, H_, row, 0, unroll=10)

    return pl.pallas_call(
        body,
        grid=(B * S,),
        in_specs=[pl.BlockSpec((FRAME_WORDS,), lambda g: (g,))],
        out_specs=pl.BlockSpec(
            (1, 1, H_, CH, W_),
            lambda g: (g // S, g % S, 0, 0, 0)),
        out_shape=jax.ShapeDtypeStruct((B, S, H_, CH, W_), jnp.float32),
    )(raw_flat)


def kernel(event_xy, event_t, event_p, event_time_range, height, width):
    del height, width
    B, S, N = event_t.shape
    F = B * S

    xy = (event_xy.reshape(B, S, N // 128, 128, 2)
          .transpose(0, 1, 2, 4, 3)
          .reshape(F * N * 2))
    t = (event_t.reshape(B, S, N // 128, 128)
         .transpose(0, 2, 1, 3).reshape(F * N))
    p = (event_p.reshape(B, S, N // 128, 128)
         .transpose(0, 2, 1, 3).reshape(F * N))

    start = event_time_range[..., 0].reshape(F)
    dur = jnp.maximum(event_time_range[..., 1].reshape(F) - start, 1.0)
    start_b = jnp.broadcast_to(start[:, None], (F, L)).reshape(F * L)
    dur_b = jnp.broadcast_to(dur[:, None], (F, L)).reshape(F * L)

    raw = _sc_histogram(xy, t, p, start_b, dur_b, F, N, S)
    out = _tc_normalize(raw, B, S)
    return jnp.transpose(out, (0, 1, 3, 2, 4))
